# Initial kernel scaffold; baseline (speedup 1.0000x reference)
import numpy as np

import jax
import jax.numpy as jnp
from jax import lax
from jax.experimental import pallas as pl

D_MODEL = 4096
N_EXPERTS = 64
TOP_K = 8
N_CAND = 32
EPS = 1e-6
TB = 256
N_TOKENS = 8192


def _tc_body(h_ref, weff_ref, wcat_ref, b_ref, pes_ref,
             probs_ref, w_ref, i_ref):
    x = h_ref[...]
    var = jnp.mean(x * x, axis=-1, keepdims=True)
    xn = x * lax.rsqrt(var + EPS) * weff_ref[...]
    logits = jnp.dot(xn, wcat_ref[...], preferred_element_type=jnp.float32)
    bvh = logits[:, :N_EXPERTS]
    rl = logits[:, N_EXPERTS:] + b_ref[...]
    m = jnp.max(rl, axis=-1, keepdims=True)
    e = jnp.exp(rl - m)
    probs = e / jnp.sum(e, axis=-1, keepdims=True)
    probs_ref[...] = probs

    v = bvh
    for _ in range(N_CAND - 1):
        mx = jnp.max(v, axis=-1, keepdims=True)
        v = jnp.where(v == mx, -jnp.inf, v)
    thresh = jnp.max(v, axis=-1, keepdims=True)
    cand_mask = bvh >= thresh

    p = jnp.where(cand_mask, probs, -1.0)
    iota = lax.broadcasted_iota(jnp.int32, (TB, N_EXPERTS), 1)
    pes = pes_ref[...]
    vals, idxs, scales = [], [], []
    for _ in range(TOP_K):
        mx = jnp.max(p, axis=-1, keepdims=True)
        sel = p == mx
        idx_r = jnp.min(jnp.where(sel, iota, N_EXPERTS), axis=-1, keepdims=True)
        pes_r = jnp.sum(jnp.where(sel, pes, 0.0), axis=-1, keepdims=True)
        vals.append(mx)
        idxs.append(idx_r)
        scales.append(pes_r)
        p = jnp.where(sel, -1.0, p)
    vals = jnp.concatenate(vals, axis=1)
    total = jnp.sum(vals, axis=-1, keepdims=True)
    w_ref[...] = vals / total * jnp.concatenate(scales, axis=1)
    i_ref[...] = jnp.concatenate(idxs, axis=1)


@jax.jit
def kernel(hidden_states, norm_weight, scale, W_proj, b_proj, W_bvh,
           per_expert_scale):
    weff = (norm_weight * scale * np.float32(np.sqrt(D_MODEL))).reshape(1, D_MODEL)
    wcat = jnp.concatenate([W_bvh, W_proj], axis=1)
    b2 = b_proj.reshape(1, N_EXPERTS)
    pes2 = per_expert_scale.reshape(1, N_EXPERTS)

    grid = (N_TOKENS // TB,)
    probs, weights, idx = pl.pallas_call(
        _tc_body,
        grid=grid,
        in_specs=[
            pl.BlockSpec((TB, D_MODEL), lambda i: (i, 0)),
            pl.BlockSpec((1, D_MODEL), lambda i: (0, 0)),
            pl.BlockSpec((D_MODEL, 2 * N_EXPERTS), lambda i: (0, 0)),
            pl.BlockSpec((1, N_EXPERTS), lambda i: (0, 0)),
            pl.BlockSpec((1, N_EXPERTS), lambda i: (0, 0)),
        ],
        out_specs=[
            pl.BlockSpec((TB, N_EXPERTS), lambda i: (i, 0)),
            pl.BlockSpec((TB, TOP_K), lambda i: (i, 0)),
            pl.BlockSpec((TB, TOP_K), lambda i: (i, 0)),
        ],
        out_shape=[
            jax.ShapeDtypeStruct((N_TOKENS, N_EXPERTS), jnp.float32),
            jax.ShapeDtypeStruct((N_TOKENS, TOP_K), jnp.float32),
            jax.ShapeDtypeStruct((N_TOKENS, TOP_K), jnp.int32),
        ],
    )(hidden_states, weff, wcat, b2, pes2)
    return probs, weights, idx


# speedup vs baseline: 1.3769x; 1.3769x over previous
import numpy as np

import jax
import jax.numpy as jnp
from jax import lax
from jax.experimental import pallas as pl

D_MODEL = 4096
N_EXPERTS = 64
TOP_K = 8
N_CAND = 32
EPS = 1e-6
TB = 256
N_TOKENS = 8192


def _tc_body(h_ref, weff_ref, wcat_ref, b_ref, pes_ref,
             probs_ref, w_ref, i_ref):
    x = h_ref[...]
    var = jnp.mean(x * x, axis=-1, keepdims=True)
    xn = x * lax.rsqrt(var + EPS) * weff_ref[...]
    logits = jnp.dot(xn, wcat_ref[...], preferred_element_type=jnp.float32)
    bvh = logits[:, :N_EXPERTS]
    rl = logits[:, N_EXPERTS:] + b_ref[...]
    m = jnp.max(rl, axis=-1, keepdims=True)
    e = jnp.exp(rl - m)
    probs = e / jnp.sum(e, axis=-1, keepdims=True)
    probs_ref[...] = probs

    v = bvh
    for _ in range(N_CAND - 1):
        mx = jnp.max(v, axis=-1, keepdims=True)
        v = jnp.where(v == mx, -jnp.inf, v)
    thresh = jnp.max(v, axis=-1, keepdims=True)
    cand_mask = bvh >= thresh

    p = jnp.where(cand_mask, probs, -1.0)
    iota = lax.broadcasted_iota(jnp.int32, (TB, N_EXPERTS), 1)
    pes = pes_ref[...]
    vals, idxs, scales = [], [], []
    for _ in range(TOP_K):
        mx = jnp.max(p, axis=-1, keepdims=True)
        sel = p == mx
        bsel = jnp.max(jnp.where(sel, bvh, -jnp.inf), axis=-1, keepdims=True)
        sel2 = jnp.logical_and(sel, bvh == bsel)
        idx_r = jnp.min(jnp.where(sel2, iota, N_EXPERTS), axis=-1, keepdims=True)
        pes_r = jnp.sum(jnp.where(sel2, pes, 0.0), axis=-1, keepdims=True)
        vals.append(mx)
        idxs.append(idx_r)
        scales.append(pes_r)
        p = jnp.where(sel2, -1.0, p)
    vals = jnp.concatenate(vals, axis=1)
    total = jnp.sum(vals, axis=-1, keepdims=True)
    w_ref[...] = vals / total * jnp.concatenate(scales, axis=1)
    i_ref[...] = jnp.concatenate(idxs, axis=1)


@jax.jit
def kernel(hidden_states, norm_weight, scale, W_proj, b_proj, W_bvh,
           per_expert_scale):
    weff = (norm_weight * scale * np.float32(np.sqrt(D_MODEL))).reshape(1, D_MODEL)
    wcat = jnp.concatenate([W_bvh, W_proj], axis=1)
    b2 = b_proj.reshape(1, N_EXPERTS)
    pes2 = per_expert_scale.reshape(1, N_EXPERTS)

    grid = (N_TOKENS // TB,)
    probs, weights, idx = pl.pallas_call(
        _tc_body,
        grid=grid,
        in_specs=[
            pl.BlockSpec((TB, D_MODEL), lambda i: (i, 0)),
            pl.BlockSpec((1, D_MODEL), lambda i: (0, 0)),
            pl.BlockSpec((D_MODEL, 2 * N_EXPERTS), lambda i: (0, 0)),
            pl.BlockSpec((1, N_EXPERTS), lambda i: (0, 0)),
            pl.BlockSpec((1, N_EXPERTS), lambda i: (0, 0)),
        ],
        out_specs=[
            pl.BlockSpec((TB, N_EXPERTS), lambda i: (i, 0)),
            pl.BlockSpec((TB, TOP_K), lambda i: (i, 0)),
            pl.BlockSpec((TB, TOP_K), lambda i: (i, 0)),
        ],
        out_shape=[
            jax.ShapeDtypeStruct((N_TOKENS, N_EXPERTS), jnp.float32),
            jax.ShapeDtypeStruct((N_TOKENS, TOP_K), jnp.float32),
            jax.ShapeDtypeStruct((N_TOKENS, TOP_K), jnp.int32),
        ],
    )(hidden_states, weff, wcat, b2, pes2)
    return probs, weights, idx


# speedup vs baseline: 1.9142x; 1.3902x over previous
import functools
import numpy as np

import jax
import jax.numpy as jnp
from jax import lax
from jax.experimental import pallas as pl
from jax.experimental.pallas import tpu as pltpu
from jax.experimental.pallas import tpu_sc as plsc

D_MODEL = 4096
N_EXPERTS = 64
TOP_K = 8
N_CAND = 32
EPS = 1e-6
TB = 256
N_TOKENS = 8192
NW = 32
TPW = N_TOKENS // NW
NEG = -3.4e38


def _tc_body(h_ref, weff_ref, wcat_ref, b_ref, probs_ref, bvh_ref):
    x = h_ref[...]
    var = jnp.mean(x * x, axis=-1, keepdims=True)
    xn = x * lax.rsqrt(var + EPS) * weff_ref[...]
    logits = jnp.dot(xn, wcat_ref[...], preferred_element_type=jnp.float32)
    bvh_ref[...] = logits[:, :N_EXPERTS]
    rl = logits[:, N_EXPERTS:] + b_ref[...]
    m = jnp.max(rl, axis=-1, keepdims=True)
    e = jnp.exp(rl - m)
    probs_ref[...] = e / jnp.sum(e, axis=-1, keepdims=True)


def _rev(x):
    return lax.rev(x, dimensions=(0,))


def _vsort(x):
    return lax.sort(x, dimension=0, is_stable=False, num_keys=1)


def _sc_body(bvh_hbm, probs_hbm, pes_hbm, w_hbm, i_hbm,
             bvh_v, probs_v, pes_v, w_v, i_v):
    wid = lax.axis_index("s") * 2 + lax.axis_index("c")
    base = wid * TPW * N_EXPERTS
    obase = wid * TPW * 16
    pltpu.sync_copy(bvh_hbm.at[pl.ds(base, TPW * N_EXPERTS)], bvh_v)
    pltpu.sync_copy(probs_hbm.at[pl.ds(base, TPW * N_EXPERTS)], probs_v)
    pltpu.sync_copy(pes_hbm, pes_v)

    iota = lax.broadcasted_iota(jnp.int32, (16,), 0)
    ids = [iota + (16 * k) for k in range(4)]

    def token_body(t, carry):
        tb = t * N_EXPERTS
        b = [bvh_v[pl.ds(tb + 16 * k, 16)] for k in range(4)]
        g = [probs_v[pl.ds(tb + 16 * k, 16)] for k in range(4)]

        s = [_vsort(b[k]) for k in range(4)]
        lo01 = _vsort(jnp.minimum(s[0], _rev(s[1])))
        hi01 = _vsort(jnp.maximum(s[0], _rev(s[1])))
        lo23 = _vsort(jnp.minimum(s[2], _rev(s[3])))
        hi23 = _vsort(jnp.maximum(s[2], _rev(s[3])))
        u0 = jnp.maximum(lo01, _rev(hi23))
        u1 = jnp.maximum(hi01, _rev(lo23))
        thr = jnp.min(jnp.minimum(u0, u1))

        p = [jnp.where(b[k] >= thr, g[k], -1.0) for k in range(4)]
        out_v = jnp.zeros((16,), jnp.float32)
        out_i = jnp.zeros((16,), jnp.int32)
        for r in range(TOP_K):
            mx = jnp.max(jnp.maximum(jnp.maximum(p[0], p[1]),
                                     jnp.maximum(p[2], p[3])))
            sel = [p[k] == mx for k in range(4)]
            tb_ = [jnp.where(sel[k], b[k], NEG) for k in range(4)]
            tm = jnp.max(jnp.maximum(jnp.maximum(tb_[0], tb_[1]),
                                     jnp.maximum(tb_[2], tb_[3])))
            sel2 = [jnp.logical_and(sel[k], b[k] == tm) for k in range(4)]
            idx = jnp.max(jnp.maximum(
                jnp.maximum(jnp.where(sel2[0], ids[0], -1),
                            jnp.where(sel2[1], ids[1], -1)),
                jnp.maximum(jnp.where(sel2[2], ids[2], -1),
                            jnp.where(sel2[3], ids[3], -1))))
            slot = iota == r
            out_v = jnp.where(slot, mx, out_v)
            out_i = jnp.where(slot, idx, out_i)
            p = [jnp.where(sel2[k], -1.0, p[k]) for k in range(4)]

        valid = iota < TOP_K
        total = jnp.sum(jnp.where(valid, out_v, 0.0))
        pes_g = plsc.load_gather(pes_v, [out_i])
        w = out_v / total * pes_g
        w_v[pl.ds(t * 16, 16)] = w
        i_v[pl.ds(t * 16, 16)] = out_i
        return carry

    lax.fori_loop(0, TPW, token_body, 0)

    pltpu.sync_copy(w_v, w_hbm.at[pl.ds(obase, TPW * 16)])
    pltpu.sync_copy(i_v, i_hbm.at[pl.ds(obase, TPW * 16)])


_sc_route = functools.partial(
    pl.kernel,
    mesh=plsc.VectorSubcoreMesh(core_axis_name="c", subcore_axis_name="s"),
    out_type=[
        jax.ShapeDtypeStruct((N_TOKENS * 16,), jnp.float32),
        jax.ShapeDtypeStruct((N_TOKENS * 16,), jnp.int32),
    ],
    scratch_types=[
        pltpu.VMEM((TPW * N_EXPERTS,), jnp.float32),
        pltpu.VMEM((TPW * N_EXPERTS,), jnp.float32),
        pltpu.VMEM((N_EXPERTS,), jnp.float32),
        pltpu.VMEM((TPW * 16,), jnp.float32),
        pltpu.VMEM((TPW * 16,), jnp.int32),
    ],
    compiler_params=pltpu.CompilerParams(needs_layout_passes=False),
)(_sc_body)


@jax.jit
def kernel(hidden_states, norm_weight, scale, W_proj, b_proj, W_bvh,
           per_expert_scale):
    weff = (norm_weight * scale * np.float32(np.sqrt(D_MODEL))).reshape(1, D_MODEL)
    wcat = jnp.concatenate([W_bvh, W_proj], axis=1)
    b2 = b_proj.reshape(1, N_EXPERTS)

    grid = (N_TOKENS // TB,)
    probs, bvh = pl.pallas_call(
        _tc_body,
        grid=grid,
        in_specs=[
            pl.BlockSpec((TB, D_MODEL), lambda i: (i, 0)),
            pl.BlockSpec((1, D_MODEL), lambda i: (0, 0)),
            pl.BlockSpec((D_MODEL, 2 * N_EXPERTS), lambda i: (0, 0)),
            pl.BlockSpec((1, N_EXPERTS), lambda i: (0, 0)),
        ],
        out_specs=[
            pl.BlockSpec((TB, N_EXPERTS), lambda i: (i, 0)),
            pl.BlockSpec((TB, N_EXPERTS), lambda i: (i, 0)),
        ],
        out_shape=[
            jax.ShapeDtypeStruct((N_TOKENS, N_EXPERTS), jnp.float32),
            jax.ShapeDtypeStruct((N_TOKENS, N_EXPERTS), jnp.float32),
        ],
    )(hidden_states, weff, wcat, b2)

    w16, i16 = _sc_route(bvh.reshape(-1), probs.reshape(-1), per_expert_scale)
    weights = w16.reshape(N_TOKENS, 16)[:, :TOP_K]
    idx = i16.reshape(N_TOKENS, 16)[:, :TOP_K]
    return probs, weights, idx
